# baseline (speedup 1.0000x reference)
import functools

import jax
import jax.numpy as jnp
from jax import lax
from jax.experimental import pallas as pl
from jax.experimental.pallas import tpu as pltpu
from jax.experimental.pallas import tpu_sc as plsc

VOCAB = 1000000
D = 64
S = 200
B = 4096
N = B * S
NC, NS = 2, 16
NW = NC * NS
BL = 128
NBB = B // BL
NCHUNK = S * NBB
PER_W = NCHUNK // NW


def _add_pos(rows_v, pos_v, s):
    iota = lax.iota(jnp.int32, 16)
    pvecs = [plsc.load_gather(pos_v, [s * D + g * 16 + iota])
             for g in range(4)]

    def rstep(r, carry):
        rvec = jnp.full((16,), r, jnp.int32)
        for g in range(4):
            dv = g * 16 + iota
            v = plsc.load_gather(rows_v, [rvec, dv])
            plsc.store_scatter(rows_v, [rvec, dv], v + pvecs[g])
        return carry

    lax.fori_loop(0, BL, rstep, 0, unroll=2)


def _body(xt_hbm, tok_hbm, pos_hbm, out_hbm,
          idx, rows, pos_v, sem_g, sem_w):
    wid = lax.axis_index("s") * NC + lax.axis_index("c")
    base = wid * PER_W
    pltpu.sync_copy(pos_hbm, pos_v)

    def stage(k, j):
        pltpu.sync_copy(xt_hbm.at[pl.ds((base + k) * BL, BL)], idx[j])
        pltpu.async_copy(tok_hbm.at[idx[j]], rows[j], sem_g[j])

    stage(0, 0)

    def pair(p, carry):
        for j in range(2):
            k = 2 * p + j

            @pl.when(k + 1 < PER_W)
            def _():
                stage(k + 1, 1 - j)

            pltpu.make_async_copy(tok_hbm.at[idx[j]], rows[j], sem_g[j]).wait()
            c = base + k
            s = c // NBB
            bb = lax.rem(c, NBB)
            _add_pos(rows[j], pos_v, s)
            pltpu.async_copy(rows[j], out_hbm.at[s, bb], sem_w)
            pltpu.make_async_copy(rows[j], out_hbm.at[s, bb], sem_w).wait()
        return carry

    lax.fori_loop(0, PER_W // 2, pair, 0)


@jax.jit
def _embed(xt_flat, tab, position_table):
    mesh = plsc.VectorSubcoreMesh(core_axis_name="c", subcore_axis_name="s")
    k = functools.partial(
        pl.kernel,
        mesh=mesh,
        out_type=jax.ShapeDtypeStruct((S, NBB, BL, D), jnp.float32),
        scratch_types=[
            [pltpu.VMEM((BL,), jnp.int32) for _ in range(2)],
            [pltpu.VMEM((BL, D), jnp.float32) for _ in range(2)],
            pltpu.VMEM((S * D,), jnp.float32),
            [pltpu.SemaphoreType.DMA for _ in range(2)],
            pltpu.SemaphoreType.DMA,
        ],
        compiler_params=pltpu.CompilerParams(
            use_tc_tiling_on_sc=False, needs_layout_passes=False),
    )(_body)
    return k(xt_flat, tab, position_table.reshape(-1))


TCB = 2048


def _detile_body(l_ref, r_ref, o_ref):
    eye = jnp.eye(D, dtype=jnp.float32)
    dn = (((0,), (0,)), ((), ()))
    o_ref[:, 0:D] = jax.lax.dot_general(
        l_ref[...], eye, dn, preferred_element_type=jnp.float32)
    o_ref[:, D:2 * D] = jax.lax.dot_general(
        r_ref[...], eye, dn, preferred_element_type=jnp.float32)


def _detile_table(token_table):
    nblk = -(-VOCAB // (2 * TCB))
    tt = jnp.pad(token_table.T, ((0, 0), (0, 2 * nblk * TCB - VOCAB)))
    packed = pl.pallas_call(
        _detile_body,
        grid=(nblk,),
        in_specs=[
            pl.BlockSpec((D, TCB), lambda i: (0, 2 * i)),
            pl.BlockSpec((D, TCB), lambda i: (0, 2 * i + 1)),
        ],
        out_specs=pl.BlockSpec((TCB, 2 * D), lambda i: (i, 0)),
        out_shape=jax.ShapeDtypeStruct((nblk * TCB, 2 * D), jnp.float32),
    )(tt, tt)
    return packed.reshape(2 * nblk * TCB, D)


def kernel(x, token_table, position_table):
    t = x.T.reshape(-1).astype(jnp.int32)
    t = ((t >> 12) << 12) | ((t & 2047) << 1) | ((t >> 11) & 1)
    tab = _detile_table(token_table)
    out4 = _embed(t, tab, position_table)
    return out4.transpose(1, 2, 0, 3).reshape(B, S, D)
